# baseline (speedup 1.0000x reference)
import functools
import jax
import jax.numpy as jnp
from jax import lax
from jax.experimental import pallas as pl
from jax.experimental.pallas import tpu as pltpu
from jax.experimental.pallas import tpu_sc as plsc

_B, _S, _D = 4, 8192, 1024
_NW = 32
_ROWS_PER_W = _S // _NW
_C = 8
_NCHUNK = _ROWS_PER_W // _C
_NBUF = 3
_LANES = 16


def _sc_body(x_hbm, emb_hbm, out_hbm, emb_v, x_v,
             in_s0, in_s1, in_s2, out_s0, out_s1, out_s2):
    in_sems = (in_s0, in_s1, in_s2)
    out_sems = (out_s0, out_s1, out_s2)
    wid = lax.axis_index("s") * 2 + lax.axis_index("c")
    row0 = wid * _ROWS_PER_W

    def in_copies(c, u):
        r = row0 + c * _C
        return (
            pltpu.make_async_copy(
                emb_hbm.at[pl.ds(r, _C)], emb_v.at[u], in_sems[u]),
            pltpu.make_async_copy(
                x_hbm.at[:, pl.ds(r, _C)], x_v.at[u], in_sems[u]),
        )

    def out_copy(c, u):
        r = row0 + c * _C
        return pltpu.make_async_copy(
            x_v.at[u], out_hbm.at[:, pl.ds(r, _C)], out_sems[u])

    def compute(u):
        for row in range(_C):
            def col_body(k, c2, row=row):
                for v in range(4):
                    col = (k * 4 + v) * _LANES
                    e = emb_v[u, row, pl.ds(col, _LANES)]
                    for b in range(_B):
                        plsc.addupdate(
                            x_v.at[u, b, row, pl.ds(col, _LANES)], e)
                return c2
            lax.fori_loop(0, _D // _LANES // 4, col_body, 0)

    def step(c, u, guard_head):
        un = (u + 1) % _NBUF
        if guard_head:
            @pl.when(c >= 2)
            def _():
                out_copy(c - 2, un).wait()
        else:
            out_copy(c - 2, un).wait()
        if guard_head or c + 1 < _NCHUNK:
            for cp in in_copies(c + 1, un):
                cp.start()
        for cp in in_copies(c, u):
            cp.wait()
        compute(u)
        out_copy(c, u).start()

    for cp in in_copies(0, 0):
        cp.start()

    def outer(i, carry):
        c0 = i * _NBUF
        for u in range(_NBUF):
            step(c0 + u, u, guard_head=True)
        return carry

    lax.fori_loop(0, (_NCHUNK - 2) // _NBUF, outer, 0)
    step(_NCHUNK - 2, (_NCHUNK - 2) % _NBUF, guard_head=False)
    u_last = (_NCHUNK - 1) % _NBUF
    out_copy(_NCHUNK - 3, (_NCHUNK - 3) % _NBUF).wait()
    for cp in in_copies(_NCHUNK - 1, u_last):
        cp.wait()
    compute(u_last)
    out_copy(_NCHUNK - 1, u_last).start()
    out_copy(_NCHUNK - 2, (_NCHUNK - 2) % _NBUF).wait()
    out_copy(_NCHUNK - 1, u_last).wait()


def kernel(x, embeddings):
    mesh = plsc.VectorSubcoreMesh(core_axis_name="c", subcore_axis_name="s")
    run = functools.partial(
        pl.kernel,
        mesh=mesh,
        out_type=jax.ShapeDtypeStruct((_B, _S, _D), jnp.float32),
        scratch_types=[
            pltpu.VMEM((_NBUF, _C, _D), jnp.float32),
            pltpu.VMEM((_NBUF, _B, _C, _D), jnp.float32),
        ] + [pltpu.SemaphoreType.DMA] * (2 * _NBUF),
    )(_sc_body)
    return run(x, embeddings)


# speedup vs baseline: 1.0664x; 1.0664x over previous
import functools
import jax
import jax.numpy as jnp
from jax import lax
from jax.experimental import pallas as pl
from jax.experimental.pallas import tpu as pltpu
from jax.experimental.pallas import tpu_sc as plsc

_B, _S, _D = 4, 8192, 1024
_NW = 32
_ROWS_PER_W = _S // _NW
_C = 4
_NCHUNK = _ROWS_PER_W // _C
_NBUF = 4
_LANES = 16
_GPR = _D // _LANES


def _sc_body(x_hbm, emb_hbm, out_hbm, emb_v, x_v,
             in_s0, in_s1, in_s2, in_s3, out_s0, out_s1, out_s2, out_s3):
    in_sems = (in_s0, in_s1, in_s2, in_s3)
    out_sems = (out_s0, out_s1, out_s2, out_s3)
    wid = lax.axis_index("s") * 2 + lax.axis_index("c")
    row0 = wid * _ROWS_PER_W

    def start_in(c, u):
        r = row0 + c * _C
        pltpu.make_async_copy(
            emb_hbm.at[pl.ds(r, _C)], emb_v.at[u], in_sems[u]).start()
        pltpu.make_async_copy(
            x_hbm.at[:, pl.ds(r, _C)], x_v.at[u], in_sems[u]).start()

    def wait_in(c, u):
        r = row0 + c * _C
        pltpu.make_async_copy(
            emb_hbm.at[pl.ds(r, _C)], emb_v.at[u], in_sems[u]).wait()
        pltpu.make_async_copy(
            x_hbm.at[:, pl.ds(r, _C)], x_v.at[u], in_sems[u]).wait()

    def start_out(c, u):
        r = row0 + c * _C
        pltpu.make_async_copy(
            x_v.at[u], out_hbm.at[:, pl.ds(r, _C)], out_sems[u]).start()

    def wait_out(c, u):
        r = row0 + c * _C
        pltpu.make_async_copy(
            x_v.at[u], out_hbm.at[:, pl.ds(r, _C)], out_sems[u]).wait()

    def compute(u):
        for row in range(_C):
            def col_body(k, c2, row=row):
                for v in range(8):
                    col = (k * 8 + v) * _LANES
                    e = emb_v[u, row, pl.ds(col, _LANES)]
                    for b in range(_B):
                        plsc.addupdate(
                            x_v.at[u, b, row, pl.ds(col, _LANES)], e)
                return c2
            lax.fori_loop(0, _GPR // 8, col_body, 0)

    start_in(0, 0)
    start_in(1, 1)

    def outer(i, carry):
        c0 = i * _NBUF
        for u in range(_NBUF):
            c = c0 + u
            uo = (u + 2) % _NBUF

            @pl.when(c >= 2)
            def _():
                wait_out(c - 2, uo)

            @pl.when(c + 2 < _NCHUNK)
            def _():
                start_in(c + 2, uo)

            wait_in(c, u)
            compute(u)
            start_out(c, u)
        return carry

    lax.fori_loop(0, _NCHUNK // _NBUF, outer, 0)
    wait_out(_NCHUNK - 2, (_NCHUNK - 2) % _NBUF)
    wait_out(_NCHUNK - 1, (_NCHUNK - 1) % _NBUF)


def kernel(x, embeddings):
    mesh = plsc.VectorSubcoreMesh(core_axis_name="c", subcore_axis_name="s")
    run = functools.partial(
        pl.kernel,
        mesh=mesh,
        out_type=jax.ShapeDtypeStruct((_B, _S, _D), jnp.float32),
        scratch_types=[
            pltpu.VMEM((_NBUF, _C, _D), jnp.float32),
            pltpu.VMEM((_NBUF, _B, _C, _D), jnp.float32),
        ] + [pltpu.SemaphoreType.DMA] * (2 * _NBUF),
    )(_sc_body)
    return run(x, embeddings)


# speedup vs baseline: 1.0772x; 1.0102x over previous
import functools
import jax
import jax.numpy as jnp
from jax import lax
from jax.experimental import pallas as pl
from jax.experimental.pallas import tpu as pltpu
from jax.experimental.pallas import tpu_sc as plsc

_B, _S, _D = 4, 8192, 1024
_NW = 32
_ROWS_PER_W = _S // _NW
_C = 2
_NCHUNK = _ROWS_PER_W // _C
_NBUF = 8
_K = 5
_LANES = 16
_GPR = _D // _LANES


def _sc_body(x_hbm, emb_hbm, out_hbm, emb_v, x_v, *sems):
    in_sems = sems[:_NBUF]
    out_sems = sems[_NBUF:]
    wid = lax.axis_index("s") * 2 + lax.axis_index("c")
    row0 = wid * _ROWS_PER_W

    def in_copies(c, u):
        r = row0 + c * _C
        return (
            pltpu.make_async_copy(
                emb_hbm.at[pl.ds(r, _C)], emb_v.at[u], in_sems[u]),
            pltpu.make_async_copy(
                x_hbm.at[:, pl.ds(r, _C)], x_v.at[u], in_sems[u]),
        )

    def out_copy(c, u):
        r = row0 + c * _C
        return pltpu.make_async_copy(
            x_v.at[u], out_hbm.at[:, pl.ds(r, _C)], out_sems[u])

    def compute(u):
        for row in range(_C):
            def col_body(k, c2, row=row):
                for v in range(4):
                    col = (k * 4 + v) * _LANES
                    e = emb_v[u, row, pl.ds(col, _LANES)]
                    for b in range(_B):
                        plsc.addupdate(
                            x_v.at[u, b, row, pl.ds(col, _LANES)], e)
                return c2
            lax.fori_loop(0, _GPR // 4, col_body, 0)

    for c in range(_K):
        for cp in in_copies(c, c % _NBUF):
            cp.start()

    def outer(i, carry):
        c0 = i * _NBUF
        for u in range(_NBUF):
            c = c0 + u
            uk = (u + _K) % _NBUF

            @pl.when(c >= _NBUF - _K)
            def _():
                out_copy(c - (_NBUF - _K), uk).wait()

            @pl.when(c + _K < _NCHUNK)
            def _():
                for cp in in_copies(c + _K, uk):
                    cp.start()

            for cp in in_copies(c, u):
                cp.wait()
            compute(u)
            out_copy(c, u).start()
        return carry

    lax.fori_loop(0, _NCHUNK // _NBUF, outer, 0)
    for c in range(_NCHUNK - (_NBUF - _K), _NCHUNK):
        out_copy(c, c % _NBUF).wait()


def kernel(x, embeddings):
    mesh = plsc.VectorSubcoreMesh(core_axis_name="c", subcore_axis_name="s")
    run = functools.partial(
        pl.kernel,
        mesh=mesh,
        out_type=jax.ShapeDtypeStruct((_B, _S, _D), jnp.float32),
        scratch_types=[
            pltpu.VMEM((_NBUF, _C, _D), jnp.float32),
            pltpu.VMEM((_NBUF, _B, _C, _D), jnp.float32),
        ] + [pltpu.SemaphoreType.DMA] * (2 * _NBUF),
    )(_sc_body)
    return run(x, embeddings)


# speedup vs baseline: 1.0793x; 1.0019x over previous
import functools
import jax
import jax.numpy as jnp
from jax import lax
from jax.experimental import pallas as pl
from jax.experimental.pallas import tpu as pltpu
from jax.experimental.pallas import tpu_sc as plsc

_B, _S, _D = 4, 8192, 1024
_NW = 32
_ROWS_PER_W = _S // _NW
_C = 2
_NCHUNK = _ROWS_PER_W // _C
_NBUF = 8
_K = 6
_LANES = 16
_GPR = _D // _LANES


def _sc_body(x_hbm, emb_hbm, out_hbm, emb_v, x_v, *sems):
    in_sems = sems[:_NBUF]
    out_sems = sems[_NBUF:]
    wid = lax.axis_index("s") * 2 + lax.axis_index("c")
    row0 = wid * _ROWS_PER_W

    def in_copies(c, u):
        r = row0 + c * _C
        return (
            pltpu.make_async_copy(
                emb_hbm.at[pl.ds(r, _C)], emb_v.at[u], in_sems[u]),
            pltpu.make_async_copy(
                x_hbm.at[:, pl.ds(r, _C)], x_v.at[u], in_sems[u]),
        )

    def out_copy(c, u):
        r = row0 + c * _C
        return pltpu.make_async_copy(
            x_v.at[u], out_hbm.at[:, pl.ds(r, _C)], out_sems[u])

    def compute(u):
        for row in range(_C):
            def col_body(k, c2, row=row):
                for v in range(4):
                    col = (k * 4 + v) * _LANES
                    e = emb_v[u, row, pl.ds(col, _LANES)]
                    for b in range(_B):
                        plsc.addupdate(
                            x_v.at[u, b, row, pl.ds(col, _LANES)], e)
                return c2
            lax.fori_loop(0, _GPR // 4, col_body, 0)

    for c in range(_K):
        for cp in in_copies(c, c % _NBUF):
            cp.start()

    def outer(i, carry):
        c0 = i * _NBUF
        for u in range(_NBUF):
            c = c0 + u
            uk = (u + _K) % _NBUF

            @pl.when(c >= _NBUF - _K)
            def _():
                out_copy(c - (_NBUF - _K), uk).wait()

            @pl.when(c + _K < _NCHUNK)
            def _():
                for cp in in_copies(c + _K, uk):
                    cp.start()

            for cp in in_copies(c, u):
                cp.wait()
            compute(u)
            out_copy(c, u).start()
        return carry

    lax.fori_loop(0, _NCHUNK // _NBUF, outer, 0)
    for c in range(_NCHUNK - (_NBUF - _K), _NCHUNK):
        out_copy(c, c % _NBUF).wait()


def kernel(x, embeddings):
    mesh = plsc.VectorSubcoreMesh(core_axis_name="c", subcore_axis_name="s")
    run = functools.partial(
        pl.kernel,
        mesh=mesh,
        out_type=jax.ShapeDtypeStruct((_B, _S, _D), jnp.float32),
        scratch_types=[
            pltpu.VMEM((_NBUF, _C, _D), jnp.float32),
            pltpu.VMEM((_NBUF, _B, _C, _D), jnp.float32),
        ] + [pltpu.SemaphoreType.DMA] * (2 * _NBUF),
    )(_sc_body)
    return run(x, embeddings)
